# baseline (speedup 1.0000x reference)
import functools

import jax
import jax.numpy as jnp
from jax import lax
from jax.experimental import pallas as pl
from jax.experimental.pallas import tpu as pltpu
from jax.experimental.pallas import tpu_sc as plsc

_NC = 2
_NS = 16
_CC = 128


def _dense_two(x, w_rel, b_rel, w_res, b_res):
    n, d = x.shape
    br = 2000 if n % 2000 == 0 else n
    grid = n // br

    def body(x_ref, wr_ref, br_ref, ws_ref, bs_ref, h_ref, r_ref):
        xv = x_ref[...]
        h_ref[...] = jnp.dot(xv, wr_ref[...], preferred_element_type=jnp.float32) + br_ref[...]
        r_ref[...] = jnp.maximum(
            jnp.dot(xv, ws_ref[...], preferred_element_type=jnp.float32) + bs_ref[...], 0.0)

    return pl.pallas_call(
        body,
        grid=(grid,),
        in_specs=[
            pl.BlockSpec((br, d), lambda i: (i, 0)),
            pl.BlockSpec((d, d), lambda i: (0, 0)),
            pl.BlockSpec((1, d), lambda i: (0, 0)),
            pl.BlockSpec((d, d), lambda i: (0, 0)),
            pl.BlockSpec((1, d), lambda i: (0, 0)),
        ],
        out_specs=[pl.BlockSpec((br, d), lambda i: (i, 0)),
                   pl.BlockSpec((br, d), lambda i: (i, 0))],
        out_shape=[jax.ShapeDtypeStruct((n, d), jnp.float32),
                   jax.ShapeDtypeStruct((n, d), jnp.float32)],
    )(x, w_rel, b_rel.reshape(1, d), w_res, b_res.reshape(1, d))


def _sc_scatter(h, idx_chunks, nchunks):
    n, d = h.shape
    n_pad = -(-(n + 1) // (_NS * 8)) * (_NS * 8)
    zpt = n_pad // _NS
    mesh = plsc.VectorSubcoreMesh(core_axis_name="c", subcore_axis_name="s",
                                  num_cores=_NC, num_subcores=_NS)

    @functools.partial(
        pl.kernel,
        out_type=jax.ShapeDtypeStruct((_NC, n_pad, d), jnp.float32),
        mesh=mesh,
        scratch_types=[
            pltpu.VMEM((2, _CC), jnp.int32),
            pltpu.VMEM((2, _CC), jnp.int32),
            pltpu.VMEM((_CC, d), jnp.float32),
            pltpu.VMEM((_CC, d), jnp.float32),
            pltpu.VMEM_SHARED((n_pad, d), jnp.float32),
            pltpu.SemaphoreType.DMA,
            pltpu.SemaphoreType.DMA,
            pltpu.SemaphoreType.DMA,
            pltpu.SemaphoreType.DMA,
        ],
    )
    def k(h_hbm, idx_hbm, out_hbm, idx0, idx1, rows0, rows1, agg_sh,
          sem_i0, sem_i1, sem_g0, sem_g1):
        c = lax.axis_index("c")
        s = lax.axis_index("s")
        wid = c * _NS + s

        nlanes = d // 16

        def zb(i, _):
            rows0[i // nlanes, pl.ds((i % nlanes) * 16, 16)] = jnp.zeros((16,), jnp.float32)
            return _

        lax.fori_loop(0, _CC * nlanes, zb, 0)

        zbase = s * zpt
        nfull, rem = zpt // _CC, zpt % _CC
        for j in range(nfull):
            pltpu.sync_copy(rows0, agg_sh.at[pl.ds(zbase + j * _CC, _CC)])
        if rem:
            pltpu.sync_copy(rows0.at[pl.ds(0, rem)], agg_sh.at[pl.ds(zbase + nfull * _CC, rem)])
        plsc.subcore_barrier()

        cid0 = wid * nchunks

        def eb(i, _):
            pltpu.sync_copy(idx_hbm.at[cid0 + i], idx0)
            pltpu.async_copy(h_hbm.at[idx0.at[0]], rows0, sem_g0).wait()
            pltpu.sync_copy(rows0, agg_sh.at[idx0.at[1]], add=True)
            return _

        lax.fori_loop(0, nchunks, eb, 0)
        plsc.subcore_barrier()

        pltpu.sync_copy(agg_sh.at[pl.ds(zbase, zpt)], out_hbm.at[c, pl.ds(zbase, zpt)])

    return k(h, idx_chunks)


def _stats(agg, res):
    n, d = res.shape
    br = 2000 if n % 2000 == 0 else n
    grid = n // br

    def body(a0_ref, a1_ref, r_ref, new_ref, st_ref):
        i = pl.program_id(0)
        new = jnp.maximum(a0_ref[0] + a1_ref[0], 0.0) + r_ref[...]
        new_ref[...] = new
        ps = jnp.concatenate(
            [jnp.sum(new, axis=0, keepdims=True),
             jnp.sum(new * new, axis=0, keepdims=True)], axis=0)

        @pl.when(i == 0)
        def _():
            st_ref[...] = ps

        @pl.when(i > 0)
        def _():
            st_ref[...] += ps

    return pl.pallas_call(
        body,
        grid=(grid,),
        in_specs=[
            pl.BlockSpec((1, br, d), lambda i: (0, i, 0)),
            pl.BlockSpec((1, br, d), lambda i: (1, i, 0)),
            pl.BlockSpec((br, d), lambda i: (i, 0)),
        ],
        out_specs=[pl.BlockSpec((br, d), lambda i: (i, 0)),
                   pl.BlockSpec((2, d), lambda i: (0, 0))],
        out_shape=[jax.ShapeDtypeStruct((n, d), jnp.float32),
                   jax.ShapeDtypeStruct((2, d), jnp.float32)],
    )(agg, agg, res)


def _bn_scale(st_ref, gamma_ref, beta_ref, n):
    ssum = st_ref[0:1, :]
    ssq = st_ref[1:2, :]
    mean = ssum * (1.0 / n)
    var = ssq * (1.0 / n) - mean * mean
    scale = gamma_ref[...] * lax.rsqrt(var + 1e-5)
    shift = beta_ref[...] - mean * scale
    return scale, shift


def _norm_dense(new, st, gamma, beta, w_rel, b_rel, w_res, b_res):
    n, d = new.shape
    br = 2000 if n % 2000 == 0 else n
    grid = n // br

    def body(new_ref, st_ref, g_ref, bt_ref, wr_ref, br_ref, ws_ref, bs_ref, h_ref, r_ref):
        scale, shift = _bn_scale(st_ref, g_ref, bt_ref, n)
        xn = new_ref[...] * scale + shift
        h_ref[...] = jnp.dot(xn, wr_ref[...], preferred_element_type=jnp.float32) + br_ref[...]
        r_ref[...] = jnp.maximum(
            jnp.dot(xn, ws_ref[...], preferred_element_type=jnp.float32) + bs_ref[...], 0.0)

    return pl.pallas_call(
        body,
        grid=(grid,),
        in_specs=[
            pl.BlockSpec((br, d), lambda i: (i, 0)),
            pl.BlockSpec((2, d), lambda i: (0, 0)),
            pl.BlockSpec((1, d), lambda i: (0, 0)),
            pl.BlockSpec((1, d), lambda i: (0, 0)),
            pl.BlockSpec((d, d), lambda i: (0, 0)),
            pl.BlockSpec((1, d), lambda i: (0, 0)),
            pl.BlockSpec((d, d), lambda i: (0, 0)),
            pl.BlockSpec((1, d), lambda i: (0, 0)),
        ],
        out_specs=[pl.BlockSpec((br, d), lambda i: (i, 0)),
                   pl.BlockSpec((br, d), lambda i: (i, 0))],
        out_shape=[jax.ShapeDtypeStruct((n, d), jnp.float32),
                   jax.ShapeDtypeStruct((n, d), jnp.float32)],
    )(new, st, gamma.reshape(1, d), beta.reshape(1, d),
      w_rel, b_rel.reshape(1, d), w_res, b_res.reshape(1, d))


def _norm(new, st, gamma, beta):
    n, d = new.shape
    br = 2000 if n % 2000 == 0 else n
    grid = n // br

    def body(new_ref, st_ref, g_ref, bt_ref, o_ref):
        scale, shift = _bn_scale(st_ref, g_ref, bt_ref, n)
        o_ref[...] = new_ref[...] * scale + shift

    return pl.pallas_call(
        body,
        grid=(grid,),
        in_specs=[
            pl.BlockSpec((br, d), lambda i: (i, 0)),
            pl.BlockSpec((2, d), lambda i: (0, 0)),
            pl.BlockSpec((1, d), lambda i: (0, 0)),
            pl.BlockSpec((1, d), lambda i: (0, 0)),
        ],
        out_specs=pl.BlockSpec((br, d), lambda i: (i, 0)),
        out_shape=jax.ShapeDtypeStruct((n, d), jnp.float32),
    )(new, st, gamma.reshape(1, d), beta.reshape(1, d))


def kernel(feats, edge_index, W1_rel, b1_rel, W1_res, b1_res, gamma1, beta1,
           W2_rel, b2_rel, W2_res, b2_res, gamma2, beta2):
    n, d = feats.shape
    e = edge_index.shape[1]
    nt = _NC * _NS
    ept = -(-e // (nt * 2 * _CC)) * (2 * _CC)
    nchunks = ept // _CC
    e_pad = nt * ept + 2 * _CC
    src = edge_index[0].astype(jnp.int32)
    dst = edge_index[1].astype(jnp.int32)
    src = jnp.concatenate([src, jnp.zeros((e_pad - e,), jnp.int32)])
    dst = jnp.concatenate([dst, jnp.full((e_pad - e,), n, jnp.int32)])
    idx_chunks = jnp.stack([src.reshape(-1, _CC), dst.reshape(-1, _CC)], axis=1)

    h1, res1 = _dense_two(feats, W1_rel, b1_rel, W1_res, b1_res)
    agg1 = _sc_scatter(h1, idx_chunks, nchunks)
    new1, st1 = _stats(agg1, res1)
    h2, res2 = _norm_dense(new1, st1, gamma1, beta1, W2_rel, b2_rel, W2_res, b2_res)
    agg2 = _sc_scatter(h2, idx_chunks, nchunks)
    new2, st2 = _stats(agg2, res2)
    return _norm(new2, st2, gamma2, beta2)


# speedup vs baseline: 1.0192x; 1.0192x over previous
import functools

import jax
import jax.numpy as jnp
from jax import lax
from jax.experimental import pallas as pl
from jax.experimental.pallas import tpu as pltpu
from jax.experimental.pallas import tpu_sc as plsc

_NC = 2
_NS = 16
_CC = 256


def _dense_two(x, w_rel, b_rel, w_res, b_res):
    n, d = x.shape
    br = 2000 if n % 2000 == 0 else n
    grid = n // br

    def body(x_ref, wr_ref, br_ref, ws_ref, bs_ref, h_ref, r_ref):
        xv = x_ref[...]
        h_ref[...] = jnp.dot(xv, wr_ref[...], preferred_element_type=jnp.float32) + br_ref[...]
        r_ref[...] = jnp.maximum(
            jnp.dot(xv, ws_ref[...], preferred_element_type=jnp.float32) + bs_ref[...], 0.0)

    return pl.pallas_call(
        body,
        grid=(grid,),
        in_specs=[
            pl.BlockSpec((br, d), lambda i: (i, 0)),
            pl.BlockSpec((d, d), lambda i: (0, 0)),
            pl.BlockSpec((1, d), lambda i: (0, 0)),
            pl.BlockSpec((d, d), lambda i: (0, 0)),
            pl.BlockSpec((1, d), lambda i: (0, 0)),
        ],
        out_specs=[pl.BlockSpec((br, d), lambda i: (i, 0)),
                   pl.BlockSpec((br, d), lambda i: (i, 0))],
        out_shape=[jax.ShapeDtypeStruct((n, d), jnp.float32),
                   jax.ShapeDtypeStruct((n, d), jnp.float32)],
    )(x, w_rel, b_rel.reshape(1, d), w_res, b_res.reshape(1, d))


def _sc_scatter(h, src_p, dst_p, ept):
    n, d = h.shape
    n_pad = -(-(n + 1) // (_NS * 8)) * (_NS * 8)
    zpt = n_pad // _NS
    nchunks = ept // _CC
    mesh = plsc.VectorSubcoreMesh(core_axis_name="c", subcore_axis_name="s",
                                  num_cores=_NC, num_subcores=_NS)

    @functools.partial(
        pl.kernel,
        out_type=jax.ShapeDtypeStruct((_NC, n_pad, d), jnp.float32),
        mesh=mesh,
        scratch_types=[
            pltpu.VMEM((_CC,), jnp.int32),
            pltpu.VMEM((_CC,), jnp.int32),
            pltpu.VMEM((_CC, d), jnp.float32),
            pltpu.VMEM_SHARED((n_pad, d), jnp.float32),
            pltpu.SemaphoreType.DMA,
        ],
    )
    def k(h_hbm, src_hbm, dst_hbm, out_hbm, src_v, dst_v, rows_v, agg_sh, sem):
        c = lax.axis_index("c")
        s = lax.axis_index("s")
        wid = c * _NS + s

        nlanes = d // 16

        def zb(i, _):
            rows_v[i // nlanes, pl.ds((i % nlanes) * 16, 16)] = jnp.zeros((16,), jnp.float32)
            return _

        lax.fori_loop(0, _CC * nlanes, zb, 0)

        zbase = s * zpt
        nfull, rem = zpt // _CC, zpt % _CC
        for j in range(nfull):
            pltpu.sync_copy(rows_v, agg_sh.at[pl.ds(zbase + j * _CC, _CC)])
        if rem:
            pltpu.sync_copy(rows_v.at[pl.ds(0, rem)], agg_sh.at[pl.ds(zbase + nfull * _CC, rem)])
        plsc.subcore_barrier()

        ebase = wid * ept

        def eb(i, _):
            off = ebase + i * _CC
            pltpu.sync_copy(src_hbm.at[pl.ds(off, _CC)], src_v)
            pltpu.sync_copy(dst_hbm.at[pl.ds(off, _CC)], dst_v)
            pltpu.async_copy(h_hbm.at[src_v], rows_v, sem).wait()
            pltpu.sync_copy(rows_v, agg_sh.at[dst_v], add=True)
            return _

        lax.fori_loop(0, nchunks, eb, 0)
        plsc.subcore_barrier()

        pltpu.sync_copy(agg_sh.at[pl.ds(zbase, zpt)], out_hbm.at[c, pl.ds(zbase, zpt)])

    return k(h, src_p, dst_p)


def _stats(agg, res):
    n, d = res.shape
    br = 2000 if n % 2000 == 0 else n
    grid = n // br

    def body(a0_ref, a1_ref, r_ref, new_ref, st_ref):
        i = pl.program_id(0)
        new = jnp.maximum(a0_ref[0] + a1_ref[0], 0.0) + r_ref[...]
        new_ref[...] = new
        ps = jnp.concatenate(
            [jnp.sum(new, axis=0, keepdims=True),
             jnp.sum(new * new, axis=0, keepdims=True)], axis=0)

        @pl.when(i == 0)
        def _():
            st_ref[...] = ps

        @pl.when(i > 0)
        def _():
            st_ref[...] += ps

    return pl.pallas_call(
        body,
        grid=(grid,),
        in_specs=[
            pl.BlockSpec((1, br, d), lambda i: (0, i, 0)),
            pl.BlockSpec((1, br, d), lambda i: (1, i, 0)),
            pl.BlockSpec((br, d), lambda i: (i, 0)),
        ],
        out_specs=[pl.BlockSpec((br, d), lambda i: (i, 0)),
                   pl.BlockSpec((2, d), lambda i: (0, 0))],
        out_shape=[jax.ShapeDtypeStruct((n, d), jnp.float32),
                   jax.ShapeDtypeStruct((2, d), jnp.float32)],
    )(agg, agg, res)


def _bn_scale(st_ref, gamma_ref, beta_ref, n):
    ssum = st_ref[0:1, :]
    ssq = st_ref[1:2, :]
    mean = ssum * (1.0 / n)
    var = ssq * (1.0 / n) - mean * mean
    scale = gamma_ref[...] * lax.rsqrt(var + 1e-5)
    shift = beta_ref[...] - mean * scale
    return scale, shift


def _norm_dense(new, st, gamma, beta, w_rel, b_rel, w_res, b_res):
    n, d = new.shape
    br = 2000 if n % 2000 == 0 else n
    grid = n // br

    def body(new_ref, st_ref, g_ref, bt_ref, wr_ref, br_ref, ws_ref, bs_ref, h_ref, r_ref):
        scale, shift = _bn_scale(st_ref, g_ref, bt_ref, n)
        xn = new_ref[...] * scale + shift
        h_ref[...] = jnp.dot(xn, wr_ref[...], preferred_element_type=jnp.float32) + br_ref[...]
        r_ref[...] = jnp.maximum(
            jnp.dot(xn, ws_ref[...], preferred_element_type=jnp.float32) + bs_ref[...], 0.0)

    return pl.pallas_call(
        body,
        grid=(grid,),
        in_specs=[
            pl.BlockSpec((br, d), lambda i: (i, 0)),
            pl.BlockSpec((2, d), lambda i: (0, 0)),
            pl.BlockSpec((1, d), lambda i: (0, 0)),
            pl.BlockSpec((1, d), lambda i: (0, 0)),
            pl.BlockSpec((d, d), lambda i: (0, 0)),
            pl.BlockSpec((1, d), lambda i: (0, 0)),
            pl.BlockSpec((d, d), lambda i: (0, 0)),
            pl.BlockSpec((1, d), lambda i: (0, 0)),
        ],
        out_specs=[pl.BlockSpec((br, d), lambda i: (i, 0)),
                   pl.BlockSpec((br, d), lambda i: (i, 0))],
        out_shape=[jax.ShapeDtypeStruct((n, d), jnp.float32),
                   jax.ShapeDtypeStruct((n, d), jnp.float32)],
    )(new, st, gamma.reshape(1, d), beta.reshape(1, d),
      w_rel, b_rel.reshape(1, d), w_res, b_res.reshape(1, d))


def _norm(new, st, gamma, beta):
    n, d = new.shape
    br = 2000 if n % 2000 == 0 else n
    grid = n // br

    def body(new_ref, st_ref, g_ref, bt_ref, o_ref):
        scale, shift = _bn_scale(st_ref, g_ref, bt_ref, n)
        o_ref[...] = new_ref[...] * scale + shift

    return pl.pallas_call(
        body,
        grid=(grid,),
        in_specs=[
            pl.BlockSpec((br, d), lambda i: (i, 0)),
            pl.BlockSpec((2, d), lambda i: (0, 0)),
            pl.BlockSpec((1, d), lambda i: (0, 0)),
            pl.BlockSpec((1, d), lambda i: (0, 0)),
        ],
        out_specs=pl.BlockSpec((br, d), lambda i: (i, 0)),
        out_shape=jax.ShapeDtypeStruct((n, d), jnp.float32),
    )(new, st, gamma.reshape(1, d), beta.reshape(1, d))


def kernel(feats, edge_index, W1_rel, b1_rel, W1_res, b1_res, gamma1, beta1,
           W2_rel, b2_rel, W2_res, b2_res, gamma2, beta2):
    n, d = feats.shape
    e = edge_index.shape[1]
    nt = _NC * _NS
    ept = -(-e // (nt * _CC)) * _CC
    e_pad = nt * ept
    src = edge_index[0].astype(jnp.int32)
    dst = edge_index[1].astype(jnp.int32)
    if e_pad > e:
        src = jnp.concatenate([src, jnp.zeros((e_pad - e,), jnp.int32)])
        dst = jnp.concatenate([dst, jnp.full((e_pad - e,), n, jnp.int32)])

    h1, res1 = _dense_two(feats, W1_rel, b1_rel, W1_res, b1_res)
    agg1 = _sc_scatter(h1, src, dst, ept)
    new1, st1 = _stats(agg1, res1)
    h2, res2 = _norm_dense(new1, st1, gamma1, beta1, W2_rel, b2_rel, W2_res, b2_res)
    agg2 = _sc_scatter(h2, src, dst, ept)
    new2, st2 = _stats(agg2, res2)
    return _norm(new2, st2, gamma2, beta2)


# speedup vs baseline: 1.4539x; 1.4265x over previous
import functools

import jax
import jax.numpy as jnp
from jax import lax
from jax.experimental import pallas as pl
from jax.experimental.pallas import tpu as pltpu
from jax.experimental.pallas import tpu_sc as plsc

_NC = 2
_NS = 16
_CC = 128


def _dense_two(x, w_rel, b_rel, w_res, b_res):
    n, d = x.shape
    br = 2000 if n % 2000 == 0 else n
    grid = n // br

    def body(x_ref, wr_ref, br_ref, ws_ref, bs_ref, h_ref, r_ref):
        xv = x_ref[...]
        h_ref[...] = jnp.dot(xv, wr_ref[...], preferred_element_type=jnp.float32) + br_ref[...]
        r_ref[...] = jnp.maximum(
            jnp.dot(xv, ws_ref[...], preferred_element_type=jnp.float32) + bs_ref[...], 0.0)

    return pl.pallas_call(
        body,
        grid=(grid,),
        in_specs=[
            pl.BlockSpec((br, d), lambda i: (i, 0)),
            pl.BlockSpec((d, d), lambda i: (0, 0)),
            pl.BlockSpec((1, d), lambda i: (0, 0)),
            pl.BlockSpec((d, d), lambda i: (0, 0)),
            pl.BlockSpec((1, d), lambda i: (0, 0)),
        ],
        out_specs=[pl.BlockSpec((br, d), lambda i: (i, 0)),
                   pl.BlockSpec((br, d), lambda i: (i, 0))],
        out_shape=[jax.ShapeDtypeStruct((n, d), jnp.float32),
                   jax.ShapeDtypeStruct((n, d), jnp.float32)],
    )(x, w_rel, b_rel.reshape(1, d), w_res, b_res.reshape(1, d))


def _dense_one(x, w, b, relu):
    n, d = x.shape
    br = 2000 if n % 2000 == 0 else n
    grid = n // br

    def body(x_ref, w_ref, b_ref, y_ref):
        y = jnp.dot(x_ref[...], w_ref[...], preferred_element_type=jnp.float32) + b_ref[...]
        y_ref[...] = jnp.maximum(y, 0.0) if relu else y

    return pl.pallas_call(
        body,
        grid=(grid,),
        in_specs=[
            pl.BlockSpec((br, d), lambda i: (i, 0)),
            pl.BlockSpec((d, d), lambda i: (0, 0)),
            pl.BlockSpec((1, d), lambda i: (0, 0)),
        ],
        out_specs=pl.BlockSpec((br, d), lambda i: (i, 0)),
        out_shape=jax.ShapeDtypeStruct((n, d), jnp.float32),
    )(x, w, b.reshape(1, d))


def _norm_dense_one(new, st, gamma, beta, w, b, relu):
    n, d = new.shape
    br = 2000 if n % 2000 == 0 else n
    grid = n // br

    def body(new_ref, st_ref, g_ref, bt_ref, w_ref, b_ref, y_ref):
        scale, shift = _bn_scale(st_ref, g_ref, bt_ref, n)
        xn = new_ref[...] * scale + shift
        y = jnp.dot(xn, w_ref[...], preferred_element_type=jnp.float32) + b_ref[...]
        y_ref[...] = jnp.maximum(y, 0.0) if relu else y

    return pl.pallas_call(
        body,
        grid=(grid,),
        in_specs=[
            pl.BlockSpec((br, d), lambda i: (i, 0)),
            pl.BlockSpec((2, d), lambda i: (0, 0)),
            pl.BlockSpec((1, d), lambda i: (0, 0)),
            pl.BlockSpec((1, d), lambda i: (0, 0)),
            pl.BlockSpec((d, d), lambda i: (0, 0)),
            pl.BlockSpec((1, d), lambda i: (0, 0)),
        ],
        out_specs=pl.BlockSpec((br, d), lambda i: (i, 0)),
        out_shape=jax.ShapeDtypeStruct((n, d), jnp.float32),
    )(new, st, gamma.reshape(1, d), beta.reshape(1, d), w, b.reshape(1, d))


def _sc_scatter(h, src_p, dst_p, ept):
    n, d = h.shape
    n_pad = -(-(n + 1) // (_NS * 8)) * (_NS * 8)
    zpt = n_pad // _NS
    nchunks = ept // _CC
    mesh = plsc.VectorSubcoreMesh(core_axis_name="c", subcore_axis_name="s",
                                  num_cores=_NC, num_subcores=_NS)

    @functools.partial(
        pl.kernel,
        out_type=jax.ShapeDtypeStruct((_NC, n_pad, d), jnp.float32),
        mesh=mesh,
        scratch_types=[
            pltpu.VMEM((_CC,), jnp.int32),
            pltpu.VMEM((_CC,), jnp.int32),
            pltpu.VMEM((_CC, d), jnp.float32),
            pltpu.VMEM_SHARED((n_pad, d), jnp.float32),
            pltpu.SemaphoreType.DMA,
        ],
    )
    def k(h_hbm, src_hbm, dst_hbm, out_hbm, src_v, dst_v, rows_v, agg_sh, sem):
        c = lax.axis_index("c")
        s = lax.axis_index("s")
        wid = c * _NS + s

        nlanes = d // 16

        def zb(i, _):
            rows_v[i // nlanes, pl.ds((i % nlanes) * 16, 16)] = jnp.zeros((16,), jnp.float32)
            return _

        lax.fori_loop(0, _CC * nlanes, zb, 0)

        zbase = s * zpt
        nfull, rem = zpt // _CC, zpt % _CC
        for j in range(nfull):
            pltpu.sync_copy(rows_v, agg_sh.at[pl.ds(zbase + j * _CC, _CC)])
        if rem:
            pltpu.sync_copy(rows_v.at[pl.ds(0, rem)], agg_sh.at[pl.ds(zbase + nfull * _CC, rem)])
        plsc.subcore_barrier()

        ebase = wid * ept

        def eb(i, _):
            off = ebase + i * _CC
            pltpu.sync_copy(src_hbm.at[pl.ds(off, _CC)], src_v)
            pltpu.sync_copy(dst_hbm.at[pl.ds(off, _CC)], dst_v)
            pltpu.async_copy(h_hbm.at[src_v], rows_v, sem).wait()
            pltpu.sync_copy(rows_v, agg_sh.at[dst_v], add=True)
            return _

        lax.fori_loop(0, nchunks, eb, 0)
        plsc.subcore_barrier()

        pltpu.sync_copy(agg_sh.at[pl.ds(zbase, zpt)], out_hbm.at[c, pl.ds(zbase, zpt)])

    return k(h, src_p, dst_p)


def _stats(agg, res):
    n, d = res.shape
    br = 2000 if n % 2000 == 0 else n
    grid = n // br

    def body(a0_ref, a1_ref, r_ref, new_ref, st_ref):
        i = pl.program_id(0)
        new = jnp.maximum(a0_ref[0] + a1_ref[0], 0.0) + r_ref[...]
        new_ref[...] = new
        ps = jnp.concatenate(
            [jnp.sum(new, axis=0, keepdims=True),
             jnp.sum(new * new, axis=0, keepdims=True)], axis=0)

        @pl.when(i == 0)
        def _():
            st_ref[...] = ps

        @pl.when(i > 0)
        def _():
            st_ref[...] += ps

    return pl.pallas_call(
        body,
        grid=(grid,),
        in_specs=[
            pl.BlockSpec((1, br, d), lambda i: (0, i, 0)),
            pl.BlockSpec((1, br, d), lambda i: (1, i, 0)),
            pl.BlockSpec((br, d), lambda i: (i, 0)),
        ],
        out_specs=[pl.BlockSpec((br, d), lambda i: (i, 0)),
                   pl.BlockSpec((2, d), lambda i: (0, 0))],
        out_shape=[jax.ShapeDtypeStruct((n, d), jnp.float32),
                   jax.ShapeDtypeStruct((2, d), jnp.float32)],
    )(agg, agg, res)


def _bn_scale(st_ref, gamma_ref, beta_ref, n):
    ssum = st_ref[0:1, :]
    ssq = st_ref[1:2, :]
    mean = ssum * (1.0 / n)
    var = ssq * (1.0 / n) - mean * mean
    scale = gamma_ref[...] * lax.rsqrt(var + 1e-5)
    shift = beta_ref[...] - mean * scale
    return scale, shift


def _norm_dense(new, st, gamma, beta, w_rel, b_rel, w_res, b_res):
    n, d = new.shape
    br = 2000 if n % 2000 == 0 else n
    grid = n // br

    def body(new_ref, st_ref, g_ref, bt_ref, wr_ref, br_ref, ws_ref, bs_ref, h_ref, r_ref):
        scale, shift = _bn_scale(st_ref, g_ref, bt_ref, n)
        xn = new_ref[...] * scale + shift
        h_ref[...] = jnp.dot(xn, wr_ref[...], preferred_element_type=jnp.float32) + br_ref[...]
        r_ref[...] = jnp.maximum(
            jnp.dot(xn, ws_ref[...], preferred_element_type=jnp.float32) + bs_ref[...], 0.0)

    return pl.pallas_call(
        body,
        grid=(grid,),
        in_specs=[
            pl.BlockSpec((br, d), lambda i: (i, 0)),
            pl.BlockSpec((2, d), lambda i: (0, 0)),
            pl.BlockSpec((1, d), lambda i: (0, 0)),
            pl.BlockSpec((1, d), lambda i: (0, 0)),
            pl.BlockSpec((d, d), lambda i: (0, 0)),
            pl.BlockSpec((1, d), lambda i: (0, 0)),
            pl.BlockSpec((d, d), lambda i: (0, 0)),
            pl.BlockSpec((1, d), lambda i: (0, 0)),
        ],
        out_specs=[pl.BlockSpec((br, d), lambda i: (i, 0)),
                   pl.BlockSpec((br, d), lambda i: (i, 0))],
        out_shape=[jax.ShapeDtypeStruct((n, d), jnp.float32),
                   jax.ShapeDtypeStruct((n, d), jnp.float32)],
    )(new, st, gamma.reshape(1, d), beta.reshape(1, d),
      w_rel, b_rel.reshape(1, d), w_res, b_res.reshape(1, d))


def _norm(new, st, gamma, beta):
    n, d = new.shape
    br = 2000 if n % 2000 == 0 else n
    grid = n // br

    def body(new_ref, st_ref, g_ref, bt_ref, o_ref):
        scale, shift = _bn_scale(st_ref, g_ref, bt_ref, n)
        o_ref[...] = new_ref[...] * scale + shift

    return pl.pallas_call(
        body,
        grid=(grid,),
        in_specs=[
            pl.BlockSpec((br, d), lambda i: (i, 0)),
            pl.BlockSpec((2, d), lambda i: (0, 0)),
            pl.BlockSpec((1, d), lambda i: (0, 0)),
            pl.BlockSpec((1, d), lambda i: (0, 0)),
        ],
        out_specs=pl.BlockSpec((br, d), lambda i: (i, 0)),
        out_shape=jax.ShapeDtypeStruct((n, d), jnp.float32),
    )(new, st, gamma.reshape(1, d), beta.reshape(1, d))


def kernel(feats, edge_index, W1_rel, b1_rel, W1_res, b1_res, gamma1, beta1,
           W2_rel, b2_rel, W2_res, b2_res, gamma2, beta2):
    n, d = feats.shape
    e = edge_index.shape[1]
    nt = _NC * _NS
    ept = -(-e // (nt * _CC)) * _CC
    e_pad = nt * ept
    src = edge_index[0].astype(jnp.int32)
    dst = edge_index[1].astype(jnp.int32)
    if e_pad > e:
        src = jnp.concatenate([src, jnp.zeros((e_pad - e,), jnp.int32)])
        dst = jnp.concatenate([dst, jnp.full((e_pad - e,), n, jnp.int32)])

    h1 = _dense_one(feats, W1_rel, b1_rel, relu=False)
    agg1 = _sc_scatter(h1, src, dst, ept)
    res1 = _dense_one(feats, W1_res, b1_res, relu=True)
    new1, st1 = _stats(agg1, res1)
    h2 = _norm_dense_one(new1, st1, gamma1, beta1, W2_rel, b2_rel, relu=False)
    agg2 = _sc_scatter(h2, src, dst, ept)
    res2 = _norm_dense_one(new1, st1, gamma1, beta1, W2_res, b2_res, relu=True)
    new2, st2 = _stats(agg2, res2)
    return _norm(new2, st2, gamma2, beta2)


# speedup vs baseline: 1.5889x; 1.0929x over previous
import functools

import jax
import jax.numpy as jnp
from jax import lax
from jax.experimental import pallas as pl
from jax.experimental.pallas import tpu as pltpu
from jax.experimental.pallas import tpu_sc as plsc

_NC = 2
_NS = 16
_CC = 128


def _dense_two(x, w_rel, b_rel, w_res, b_res):
    n, d = x.shape
    br = 2000 if n % 2000 == 0 else n
    grid = n // br

    def body(x_ref, wr_ref, br_ref, ws_ref, bs_ref, h_ref, r_ref):
        xv = x_ref[...]
        h_ref[...] = jnp.dot(xv, wr_ref[...], preferred_element_type=jnp.float32) + br_ref[...]
        r_ref[...] = jnp.maximum(
            jnp.dot(xv, ws_ref[...], preferred_element_type=jnp.float32) + bs_ref[...], 0.0)

    return pl.pallas_call(
        body,
        grid=(grid,),
        in_specs=[
            pl.BlockSpec((br, d), lambda i: (i, 0)),
            pl.BlockSpec((d, d), lambda i: (0, 0)),
            pl.BlockSpec((1, d), lambda i: (0, 0)),
            pl.BlockSpec((d, d), lambda i: (0, 0)),
            pl.BlockSpec((1, d), lambda i: (0, 0)),
        ],
        out_specs=[pl.BlockSpec((br, d), lambda i: (i, 0)),
                   pl.BlockSpec((br, d), lambda i: (i, 0))],
        out_shape=[jax.ShapeDtypeStruct((n, d), jnp.float32),
                   jax.ShapeDtypeStruct((n, d), jnp.float32)],
    )(x, w_rel, b_rel.reshape(1, d), w_res, b_res.reshape(1, d))


def _dense_one(x, w, b, relu):
    n, d = x.shape
    br = 2000 if n % 2000 == 0 else n
    grid = n // br

    def body(x_ref, w_ref, b_ref, y_ref):
        y = jnp.dot(x_ref[...], w_ref[...], preferred_element_type=jnp.float32) + b_ref[...]
        y_ref[...] = jnp.maximum(y, 0.0) if relu else y

    return pl.pallas_call(
        body,
        grid=(grid,),
        in_specs=[
            pl.BlockSpec((br, d), lambda i: (i, 0)),
            pl.BlockSpec((d, d), lambda i: (0, 0)),
            pl.BlockSpec((1, d), lambda i: (0, 0)),
        ],
        out_specs=pl.BlockSpec((br, d), lambda i: (i, 0)),
        out_shape=jax.ShapeDtypeStruct((n, d), jnp.float32),
    )(x, w, b.reshape(1, d))


def _norm_dense_one(new, st, gamma, beta, w, b, relu):
    n, d = new.shape
    br = 2000 if n % 2000 == 0 else n
    grid = n // br

    def body(new_ref, st_ref, g_ref, bt_ref, w_ref, b_ref, y_ref):
        scale, shift = _bn_scale(st_ref, g_ref, bt_ref, n)
        xn = new_ref[...] * scale + shift
        y = jnp.dot(xn, w_ref[...], preferred_element_type=jnp.float32) + b_ref[...]
        y_ref[...] = jnp.maximum(y, 0.0) if relu else y

    return pl.pallas_call(
        body,
        grid=(grid,),
        in_specs=[
            pl.BlockSpec((br, d), lambda i: (i, 0)),
            pl.BlockSpec((2, d), lambda i: (0, 0)),
            pl.BlockSpec((1, d), lambda i: (0, 0)),
            pl.BlockSpec((1, d), lambda i: (0, 0)),
            pl.BlockSpec((d, d), lambda i: (0, 0)),
            pl.BlockSpec((1, d), lambda i: (0, 0)),
        ],
        out_specs=pl.BlockSpec((br, d), lambda i: (i, 0)),
        out_shape=jax.ShapeDtypeStruct((n, d), jnp.float32),
    )(new, st, gamma.reshape(1, d), beta.reshape(1, d), w, b.reshape(1, d))


def _sc_scatter(h, src_p, dst_p, ept):
    n, d = h.shape
    n_pad = -(-(n + 1) // (_NS * 8)) * (_NS * 8)
    zpt = n_pad // _NS
    nchunks = ept // _CC
    mesh = plsc.VectorSubcoreMesh(core_axis_name="c", subcore_axis_name="s",
                                  num_cores=_NC, num_subcores=_NS)

    @functools.partial(
        pl.kernel,
        out_type=jax.ShapeDtypeStruct((_NC, n_pad, d), jnp.float32),
        mesh=mesh,
        scratch_types=[
            pltpu.VMEM((ept,), jnp.int32),
            pltpu.VMEM((_CC,), jnp.int32),
            pltpu.VMEM((_CC, d), jnp.float32),
            pltpu.VMEM_SHARED((n_pad, d), jnp.float32),
            pltpu.SemaphoreType.DMA,
        ],
    )
    def k(h_hbm, src_hbm, dst_hbm, out_hbm, src_v, dst_v, rows_v, agg_sh, sem):
        c = lax.axis_index("c")
        s = lax.axis_index("s")
        wid = c * _NS + s

        nlanes = d // 16

        def zb(i, _):
            rows_v[i // nlanes, pl.ds((i % nlanes) * 16, 16)] = jnp.zeros((16,), jnp.float32)
            return _

        lax.fori_loop(0, _CC * nlanes, zb, 0)

        zbase = s * zpt
        nfull, rem = zpt // _CC, zpt % _CC
        for j in range(nfull):
            pltpu.sync_copy(rows_v, agg_sh.at[pl.ds(zbase + j * _CC, _CC)])
        if rem:
            pltpu.sync_copy(rows_v.at[pl.ds(0, rem)], agg_sh.at[pl.ds(zbase + nfull * _CC, rem)])
        plsc.subcore_barrier()

        ebase = wid * ept
        pltpu.sync_copy(src_hbm.at[pl.ds(ebase, ept)], src_v)

        def eb(i, _):
            pltpu.sync_copy(dst_hbm.at[pl.ds(ebase + i * _CC, _CC)], dst_v)
            pltpu.async_copy(h_hbm.at[src_v.at[pl.ds(i * _CC, _CC)]], rows_v, sem).wait()
            pltpu.sync_copy(rows_v, agg_sh.at[dst_v], add=True)
            return _

        lax.fori_loop(0, nchunks, eb, 0)
        plsc.subcore_barrier()

        pltpu.sync_copy(agg_sh.at[pl.ds(zbase, zpt)], out_hbm.at[c, pl.ds(zbase, zpt)])

    return k(h, src_p, dst_p)


def _stats(agg, res):
    n, d = res.shape
    br = 2000 if n % 2000 == 0 else n
    grid = n // br

    def body(a0_ref, a1_ref, r_ref, new_ref, st_ref):
        i = pl.program_id(0)
        new = jnp.maximum(a0_ref[0] + a1_ref[0], 0.0) + r_ref[...]
        new_ref[...] = new
        ps = jnp.concatenate(
            [jnp.sum(new, axis=0, keepdims=True),
             jnp.sum(new * new, axis=0, keepdims=True)], axis=0)

        @pl.when(i == 0)
        def _():
            st_ref[...] = ps

        @pl.when(i > 0)
        def _():
            st_ref[...] += ps

    return pl.pallas_call(
        body,
        grid=(grid,),
        in_specs=[
            pl.BlockSpec((1, br, d), lambda i: (0, i, 0)),
            pl.BlockSpec((1, br, d), lambda i: (1, i, 0)),
            pl.BlockSpec((br, d), lambda i: (i, 0)),
        ],
        out_specs=[pl.BlockSpec((br, d), lambda i: (i, 0)),
                   pl.BlockSpec((2, d), lambda i: (0, 0))],
        out_shape=[jax.ShapeDtypeStruct((n, d), jnp.float32),
                   jax.ShapeDtypeStruct((2, d), jnp.float32)],
    )(agg, agg, res)


def _bn_scale(st_ref, gamma_ref, beta_ref, n):
    ssum = st_ref[0:1, :]
    ssq = st_ref[1:2, :]
    mean = ssum * (1.0 / n)
    var = ssq * (1.0 / n) - mean * mean
    scale = gamma_ref[...] * lax.rsqrt(var + 1e-5)
    shift = beta_ref[...] - mean * scale
    return scale, shift


def _norm_dense(new, st, gamma, beta, w_rel, b_rel, w_res, b_res):
    n, d = new.shape
    br = 2000 if n % 2000 == 0 else n
    grid = n // br

    def body(new_ref, st_ref, g_ref, bt_ref, wr_ref, br_ref, ws_ref, bs_ref, h_ref, r_ref):
        scale, shift = _bn_scale(st_ref, g_ref, bt_ref, n)
        xn = new_ref[...] * scale + shift
        h_ref[...] = jnp.dot(xn, wr_ref[...], preferred_element_type=jnp.float32) + br_ref[...]
        r_ref[...] = jnp.maximum(
            jnp.dot(xn, ws_ref[...], preferred_element_type=jnp.float32) + bs_ref[...], 0.0)

    return pl.pallas_call(
        body,
        grid=(grid,),
        in_specs=[
            pl.BlockSpec((br, d), lambda i: (i, 0)),
            pl.BlockSpec((2, d), lambda i: (0, 0)),
            pl.BlockSpec((1, d), lambda i: (0, 0)),
            pl.BlockSpec((1, d), lambda i: (0, 0)),
            pl.BlockSpec((d, d), lambda i: (0, 0)),
            pl.BlockSpec((1, d), lambda i: (0, 0)),
            pl.BlockSpec((d, d), lambda i: (0, 0)),
            pl.BlockSpec((1, d), lambda i: (0, 0)),
        ],
        out_specs=[pl.BlockSpec((br, d), lambda i: (i, 0)),
                   pl.BlockSpec((br, d), lambda i: (i, 0))],
        out_shape=[jax.ShapeDtypeStruct((n, d), jnp.float32),
                   jax.ShapeDtypeStruct((n, d), jnp.float32)],
    )(new, st, gamma.reshape(1, d), beta.reshape(1, d),
      w_rel, b_rel.reshape(1, d), w_res, b_res.reshape(1, d))


def _norm(new, st, gamma, beta):
    n, d = new.shape
    br = 2000 if n % 2000 == 0 else n
    grid = n // br

    def body(new_ref, st_ref, g_ref, bt_ref, o_ref):
        scale, shift = _bn_scale(st_ref, g_ref, bt_ref, n)
        o_ref[...] = new_ref[...] * scale + shift

    return pl.pallas_call(
        body,
        grid=(grid,),
        in_specs=[
            pl.BlockSpec((br, d), lambda i: (i, 0)),
            pl.BlockSpec((2, d), lambda i: (0, 0)),
            pl.BlockSpec((1, d), lambda i: (0, 0)),
            pl.BlockSpec((1, d), lambda i: (0, 0)),
        ],
        out_specs=pl.BlockSpec((br, d), lambda i: (i, 0)),
        out_shape=jax.ShapeDtypeStruct((n, d), jnp.float32),
    )(new, st, gamma.reshape(1, d), beta.reshape(1, d))


def kernel(feats, edge_index, W1_rel, b1_rel, W1_res, b1_res, gamma1, beta1,
           W2_rel, b2_rel, W2_res, b2_res, gamma2, beta2):
    n, d = feats.shape
    e = edge_index.shape[1]
    nt = _NC * _NS
    ept = -(-e // (nt * _CC)) * _CC
    e_pad = nt * ept
    src = edge_index[0].astype(jnp.int32)
    dst = edge_index[1].astype(jnp.int32)
    if e_pad > e:
        src = jnp.concatenate([src, jnp.zeros((e_pad - e,), jnp.int32)])
        dst = jnp.concatenate([dst, jnp.full((e_pad - e,), n, jnp.int32)])

    h1 = _dense_one(feats, W1_rel, b1_rel, relu=False)
    agg1 = _sc_scatter(h1, src, dst, ept)
    res1 = _dense_one(feats, W1_res, b1_res, relu=True)
    new1, st1 = _stats(agg1, res1)
    h2 = _norm_dense_one(new1, st1, gamma1, beta1, W2_rel, b2_rel, relu=False)
    agg2 = _sc_scatter(h2, src, dst, ept)
    res2 = _norm_dense_one(new1, st1, gamma1, beta1, W2_res, b2_res, relu=True)
    new2, st2 = _stats(agg2, res2)
    return _norm(new2, st2, gamma2, beta2)
